# baseline (speedup 1.0000x reference)
import functools

import jax
import jax.numpy as jnp
from jax import lax
from jax.experimental import pallas as pl
from jax.experimental.pallas import tpu as pltpu
from jax.experimental.pallas import tpu_sc as plsc

_NC = 2
_NS = 16
_NW = _NC * _NS
_L = 16
_CHP = 16
_K = 80
_UNROLL = 16


def _sc_edge_pass(a_tab, b_tab, src3, dst3, w1p, n_nodes, ch):
    nchunk = src3.shape[1]
    rpt = n_nodes // _NS

    mesh = plsc.VectorSubcoreMesh(core_axis_name="c", subcore_axis_name="s",
                                  num_cores=_NC, num_subcores=_NS)

    @functools.partial(
        pl.kernel,
        out_type=jax.ShapeDtypeStruct((_NW, rpt, _CHP), jnp.float32),
        mesh=mesh,
        compiler_params=pltpu.CompilerParams(use_tc_tiling_on_sc=False),
        scratch_types=[
            pltpu.VMEM((nchunk, _K), jnp.int32),
            pltpu.VMEM((nchunk, _K), jnp.int32),
            pltpu.VMEM((2, _K, _CHP), jnp.float32),
            pltpu.VMEM((2, _K, _CHP), jnp.float32),
            pltpu.VMEM((2, _K, _CHP), jnp.float32),
            pltpu.VMEM((_CHP, _CHP), jnp.float32),
            pltpu.VMEM((rpt, _CHP), jnp.float32),
            pltpu.VMEM_SHARED((n_nodes, _CHP), jnp.float32),
            pltpu.SemaphoreType.DMA,
            pltpu.SemaphoreType.DMA,
            pltpu.SemaphoreType.DMA,
            pltpu.SemaphoreType.DMA,
            pltpu.SemaphoreType.DMA,
            pltpu.SemaphoreType.DMA,
        ],
    )
    def edge_kernel(a_hbm, b_hbm, src_hbm, dst_hbm, w1_hbm, out_hbm,
                    sidx, didx, abuf, bbuf, mbuf, w1v, zbuf, agg,
                    sem_a0, sem_b0, sem_a1, sem_b1, sem_s0, sem_s1):
        sems = ((sem_a0, sem_b0), (sem_a1, sem_b1))
        ssems = (sem_s0, sem_s1)
        cid = lax.axis_index("c")
        tid = lax.axis_index("s")
        wid = cid * _NS + tid

        zv = jnp.zeros((_L,), jnp.float32)

        def _zero_z(i, carry):
            zbuf[i, :] = zv
            return carry

        lax.fori_loop(0, rpt, _zero_z, 0)

        def _zero_m(i, carry):
            mbuf[0, i, :] = zv
            mbuf[1, i, :] = zv
            return carry

        lax.fori_loop(0, _K, _zero_m, 0)

        pltpu.sync_copy(zbuf, agg.at[pl.ds(tid * rpt, rpt)])
        pltpu.sync_copy(w1_hbm, w1v)
        pltpu.sync_copy(src_hbm.at[wid], sidx)
        pltpu.sync_copy(dst_hbm.at[wid], didx)
        plsc.subcore_barrier()

        w1r = [w1v[c, :] for c in range(ch)]
        bcidx = [jnp.full((_L,), c, jnp.int32) for c in range(ch)]

        def issue_gathers(g, slot):
            pltpu.async_copy(a_hbm.at[didx.at[g]], abuf.at[slot], sems[slot][0])
            pltpu.async_copy(b_hbm.at[sidx.at[g]], bbuf.at[slot], sems[slot][1])

        def wait_gathers(g, slot):
            pltpu.make_async_copy(a_hbm.at[didx.at[g]], abuf.at[slot],
                                  sems[slot][0]).wait()
            pltpu.make_async_copy(b_hbm.at[sidx.at[g]], bbuf.at[slot],
                                  sems[slot][1]).wait()

        def wait_scatter(slot):
            pltpu.make_async_copy(mbuf.at[slot], agg.at[didx.at[0]],
                                  ssems[slot]).wait()

        def compute_scatter(g, slot):
            @plsc.parallel_loop(0, _K, unroll=_UNROLL)
            def edge_body(e):
                m = jnp.maximum(abuf[slot, e, :] + bbuf[slot, e, :], 0.0)
                terms = [m.at[bcidx[ci]].get(mode="promise_in_bounds")
                         * w1r[ci] for ci in range(ch)]
                while len(terms) > 1:
                    terms = [terms[i] + terms[i + 1]
                             for i in range(0, len(terms) - 1, 2)] + (
                                 [terms[-1]] if len(terms) % 2 else [])
                mbuf[slot, e, :] = jnp.maximum(terms[0], 0.0)

            pltpu.async_copy(mbuf.at[slot], agg.at[didx.at[g]], ssems[slot],
                             add=True)

        issue_gathers(0, 0)
        pltpu.async_copy(mbuf.at[0], agg.at[didx.at[0]], ssems[0], add=True)
        pltpu.async_copy(mbuf.at[1], agg.at[didx.at[0]], ssems[1], add=True)

        def chunk_pair(i, carry):
            g0 = 2 * i
            issue_gathers(g0 + 1, 1)
            wait_gathers(g0, 0)
            wait_scatter(0)
            compute_scatter(g0, 0)
            issue_gathers(g0 + 2, 0)
            wait_gathers(g0 + 1, 1)
            wait_scatter(1)
            compute_scatter(g0 + 1, 1)
            return carry

        lax.fori_loop(0, (nchunk - 1) // 2, chunk_pair, 0)
        wait_gathers(nchunk - 1, 0)
        wait_scatter(0)
        compute_scatter(nchunk - 1, 0)
        wait_scatter(0)
        wait_scatter(1)

        plsc.subcore_barrier()
        pltpu.sync_copy(agg.at[pl.ds(tid * rpt, rpt)], out_hbm.at[wid])

    return edge_kernel(a_tab, b_tab, src3, dst3, w1p)


def _tc_pre(x, wt, wb):
    n = x.shape[0]

    def body(x_ref, wt_ref, wb_ref, a_ref, b_ref):
        xv = x_ref[...]
        a_ref[...] = jnp.dot(xv, wt_ref[...], preferred_element_type=jnp.float32)
        b_ref[...] = jnp.dot(xv, wb_ref[...], preferred_element_type=jnp.float32)

    return pl.pallas_call(
        body,
        out_shape=[jax.ShapeDtypeStruct((n, _CHP), jnp.float32)] * 2,
    )(x, wt, wb)


def _tc_mid(parts, wt, wb):
    n = parts.shape[1]

    def body(p_ref, wt_ref, wb_ref, a_ref, b_ref):
        h = jnp.maximum(p_ref[0] + p_ref[1], 0.0)
        a_ref[...] = jnp.dot(h, wt_ref[...], preferred_element_type=jnp.float32)
        b_ref[...] = jnp.dot(h, wb_ref[...], preferred_element_type=jnp.float32)

    return pl.pallas_call(
        body,
        out_shape=[jax.ShapeDtypeStruct((n, _CHP), jnp.float32)] * 2,
    )(parts, wt, wb)


def _tc_fin(parts, wf1p, wf2):
    n = parts.shape[1]
    feat = wf2.shape[1]

    def body(p_ref, w1_ref, w2_ref, o_ref):
        h = jnp.maximum(p_ref[0] + p_ref[1], 0.0)
        t = jnp.maximum(jnp.dot(h, w1_ref[...], preferred_element_type=jnp.float32), 0.0)
        o_ref[...] = jnp.maximum(jnp.dot(t, w2_ref[...], preferred_element_type=jnp.float32), 0.0)

    return pl.pallas_call(
        body,
        out_shape=jax.ShapeDtypeStruct((n, feat), jnp.float32),
    )(parts, wf1p, wf2)


def kernel(x, edge_index, edge_attr, W2_0, W1_0, W2_1, W1_1, Wf1, Wf2):
    del edge_attr
    n, feat = x.shape
    ch = W1_0.shape[0]
    pad_c = _CHP - ch

    n_edges = edge_index.shape[1]
    nchunk = n_edges // (_NW * _K)
    src3 = edge_index[0].astype(jnp.int32).reshape(_NW, nchunk, _K)
    dst3 = edge_index[1].astype(jnp.int32).reshape(_NW, nchunk, _K)

    wt0 = jnp.pad(W2_0[:feat], ((0, 0), (0, pad_c)))
    wb0 = jnp.pad(W2_0[feat:], ((0, 0), (0, pad_c)))
    w1p0 = jnp.pad(W1_0 * 0.1, ((0, pad_c), (0, pad_c)))
    wt1 = jnp.pad(W2_1[:ch], ((0, pad_c), (0, pad_c)))
    wb1 = jnp.pad(W2_1[ch:], ((0, pad_c), (0, pad_c)))
    w1p1 = jnp.pad(W1_1 * 0.1, ((0, pad_c), (0, pad_c)))
    wf1p = jnp.pad(Wf1, ((0, pad_c), (0, 0)))

    a0, b0 = _tc_pre(x, wt0, wb0)
    parts0 = _sc_edge_pass(a0, b0, src3, dst3, w1p0, n, ch)
    a1, b1 = _tc_mid(parts0.reshape(_NC, n, _CHP), wt1, wb1)
    parts1 = _sc_edge_pass(a1, b1, src3, dst3, w1p1, n, ch)
    return _tc_fin(parts1.reshape(_NC, n, _CHP), wf1p, Wf2)


# speedup vs baseline: 2.2425x; 2.2425x over previous
import functools

import jax
import jax.numpy as jnp
from jax import lax
from jax.experimental import pallas as pl
from jax.experimental.pallas import tpu as pltpu
from jax.experimental.pallas import tpu_sc as plsc

_NC = 2
_NS = 16
_NW = _NC * _NS
_L = 16
_CHP = 16
_K = 80
_UNROLL = 8


def _sc_edge_pass(a_tab, b_tab, src3, dst3, w1p, n_nodes, ch):
    nchunk = src3.shape[1]
    rpt = n_nodes // _NS

    mesh = plsc.VectorSubcoreMesh(core_axis_name="c", subcore_axis_name="s",
                                  num_cores=_NC, num_subcores=_NS)

    @functools.partial(
        pl.kernel,
        out_type=jax.ShapeDtypeStruct((_NW, rpt, _CHP), jnp.float32),
        mesh=mesh,
        compiler_params=pltpu.CompilerParams(use_tc_tiling_on_sc=False),
        scratch_types=[
            pltpu.VMEM((nchunk, _K), jnp.int32),
            pltpu.VMEM((nchunk, _K), jnp.int32),
            pltpu.VMEM((2, _K, _CHP), jnp.float32),
            pltpu.VMEM((2, _K, _CHP), jnp.float32),
            pltpu.VMEM((2, _K, _CHP), jnp.float32),
            pltpu.VMEM((_CHP, _CHP), jnp.float32),
            pltpu.VMEM((rpt, _CHP), jnp.float32),
            pltpu.VMEM_SHARED((n_nodes, _CHP), jnp.float32),
            pltpu.SemaphoreType.DMA,
            pltpu.SemaphoreType.DMA,
            pltpu.SemaphoreType.DMA,
            pltpu.SemaphoreType.DMA,
            pltpu.SemaphoreType.DMA,
            pltpu.SemaphoreType.DMA,
        ],
    )
    def edge_kernel(a_hbm, b_hbm, src_hbm, dst_hbm, w1_hbm, out_hbm,
                    sidx, didx, abuf, bbuf, mbuf, w1v, zbuf, agg,
                    sem_a0, sem_b0, sem_a1, sem_b1, sem_s0, sem_s1):
        sems = ((sem_a0, sem_b0), (sem_a1, sem_b1))
        ssems = (sem_s0, sem_s1)
        cid = lax.axis_index("c")
        tid = lax.axis_index("s")
        wid = cid * _NS + tid

        zv = jnp.zeros((_L,), jnp.float32)

        def _zero_z(i, carry):
            zbuf[i, :] = zv
            return carry

        lax.fori_loop(0, rpt, _zero_z, 0)

        def _zero_m(i, carry):
            mbuf[0, i, :] = zv
            mbuf[1, i, :] = zv
            return carry

        lax.fori_loop(0, _K, _zero_m, 0)

        pltpu.sync_copy(zbuf, agg.at[pl.ds(tid * rpt, rpt)])
        pltpu.sync_copy(w1_hbm, w1v)
        pltpu.sync_copy(src_hbm.at[wid], sidx)
        pltpu.sync_copy(dst_hbm.at[wid], didx)
        plsc.subcore_barrier()

        w1r = [w1v[c, :] for c in range(ch)]
        bcidx = [jnp.full((_L,), c, jnp.int32) for c in range(ch)]

        def issue_gathers(g, slot):
            pltpu.async_copy(a_hbm.at[didx.at[g]], abuf.at[slot], sems[slot][0])
            pltpu.async_copy(b_hbm.at[sidx.at[g]], bbuf.at[slot], sems[slot][1])

        def wait_gathers(g, slot):
            pltpu.make_async_copy(a_hbm.at[didx.at[g]], abuf.at[slot],
                                  sems[slot][0]).wait()
            pltpu.make_async_copy(b_hbm.at[sidx.at[g]], bbuf.at[slot],
                                  sems[slot][1]).wait()

        def wait_scatter(slot):
            pltpu.make_async_copy(mbuf.at[slot], agg.at[didx.at[0]],
                                  ssems[slot]).wait()

        def compute_scatter(g, slot):
            @plsc.parallel_loop(0, _K, unroll=_UNROLL)
            def edge_body(e):
                m = jnp.maximum(abuf[slot, e, :] + bbuf[slot, e, :], 0.0)
                terms = [m.at[bcidx[ci]].get(mode="promise_in_bounds")
                         * w1r[ci] for ci in range(ch)]
                while len(terms) > 1:
                    terms = [terms[i] + terms[i + 1]
                             for i in range(0, len(terms) - 1, 2)] + (
                                 [terms[-1]] if len(terms) % 2 else [])
                mbuf[slot, e, :] = jnp.maximum(terms[0], 0.0)

            pltpu.async_copy(mbuf.at[slot], agg.at[didx.at[g]], ssems[slot],
                             add=True)

        issue_gathers(0, 0)
        pltpu.async_copy(mbuf.at[0], agg.at[didx.at[0]], ssems[0], add=True)
        pltpu.async_copy(mbuf.at[1], agg.at[didx.at[0]], ssems[1], add=True)

        def chunk_pair(i, carry):
            g0 = 2 * i
            issue_gathers(g0 + 1, 1)
            wait_gathers(g0, 0)
            wait_scatter(0)
            compute_scatter(g0, 0)
            issue_gathers(g0 + 2, 0)
            wait_gathers(g0 + 1, 1)
            wait_scatter(1)
            compute_scatter(g0 + 1, 1)
            return carry

        lax.fori_loop(0, (nchunk - 1) // 2, chunk_pair, 0)
        wait_gathers(nchunk - 1, 0)
        wait_scatter(0)
        compute_scatter(nchunk - 1, 0)
        wait_scatter(0)
        wait_scatter(1)

        plsc.subcore_barrier()
        pltpu.sync_copy(agg.at[pl.ds(tid * rpt, rpt)], out_hbm.at[wid])

    return edge_kernel(a_tab, b_tab, src3, dst3, w1p)


def _tc_pre(x, wt, wb):
    n = x.shape[0]

    def body(x_ref, wt_ref, wb_ref, a_ref, b_ref):
        xv = x_ref[...]
        a_ref[...] = jnp.dot(xv, wt_ref[...], preferred_element_type=jnp.float32)
        b_ref[...] = jnp.dot(xv, wb_ref[...], preferred_element_type=jnp.float32)

    return pl.pallas_call(
        body,
        out_shape=[jax.ShapeDtypeStruct((n, _CHP), jnp.float32)] * 2,
    )(x, wt, wb)


def _tc_mid(parts, wt, wb):
    n = parts.shape[1]

    def body(p_ref, wt_ref, wb_ref, a_ref, b_ref):
        h = jnp.maximum(p_ref[0] + p_ref[1], 0.0)
        a_ref[...] = jnp.dot(h, wt_ref[...], preferred_element_type=jnp.float32)
        b_ref[...] = jnp.dot(h, wb_ref[...], preferred_element_type=jnp.float32)

    return pl.pallas_call(
        body,
        out_shape=[jax.ShapeDtypeStruct((n, _CHP), jnp.float32)] * 2,
    )(parts, wt, wb)


def _tc_fin(parts, wf1p, wf2):
    n = parts.shape[1]
    feat = wf2.shape[1]

    def body(p_ref, w1_ref, w2_ref, o_ref):
        h = jnp.maximum(p_ref[0] + p_ref[1], 0.0)
        t = jnp.maximum(jnp.dot(h, w1_ref[...], preferred_element_type=jnp.float32), 0.0)
        o_ref[...] = jnp.maximum(jnp.dot(t, w2_ref[...], preferred_element_type=jnp.float32), 0.0)

    return pl.pallas_call(
        body,
        out_shape=jax.ShapeDtypeStruct((n, feat), jnp.float32),
    )(parts, wf1p, wf2)


def kernel(x, edge_index, edge_attr, W2_0, W1_0, W2_1, W1_1, Wf1, Wf2):
    del edge_attr
    n, feat = x.shape
    ch = W1_0.shape[0]
    pad_c = _CHP - ch

    n_edges = edge_index.shape[1]
    nchunk = n_edges // (_NW * _K)
    src3 = edge_index[0].astype(jnp.int32).reshape(_NW, nchunk, _K)
    dst3 = edge_index[1].astype(jnp.int32).reshape(_NW, nchunk, _K)

    wt0 = jnp.pad(W2_0[:feat], ((0, 0), (0, pad_c)))
    wb0 = jnp.pad(W2_0[feat:], ((0, 0), (0, pad_c)))
    w1p0 = jnp.pad(W1_0 * 0.1, ((0, pad_c), (0, pad_c)))
    wt1 = jnp.pad(W2_1[:ch], ((0, pad_c), (0, pad_c)))
    wb1 = jnp.pad(W2_1[ch:], ((0, pad_c), (0, pad_c)))
    w1p1 = jnp.pad(W1_1 * 0.1, ((0, pad_c), (0, pad_c)))
    wf1p = jnp.pad(Wf1, ((0, pad_c), (0, 0)))

    a0, b0 = _tc_pre(x, wt0, wb0)
    parts0 = _sc_edge_pass(a0, b0, src3, dst3, w1p0, n, ch)
    a1, b1 = _tc_mid(parts0.reshape(_NC, n, _CHP), wt1, wb1)
    parts1 = _sc_edge_pass(a1, b1, src3, dst3, w1p1, n, ch)
    return _tc_fin(parts1.reshape(_NC, n, _CHP), wf1p, Wf2)


# speedup vs baseline: 2.3505x; 1.0482x over previous
import functools

import jax
import jax.numpy as jnp
from jax import lax
from jax.experimental import pallas as pl
from jax.experimental.pallas import tpu as pltpu
from jax.experimental.pallas import tpu_sc as plsc

_NC = 2
_NS = 16
_NW = _NC * _NS
_L = 16
_CHP = 16
_K = 80
_UNROLL = 8


def _sc_edge_pass(a_tab, b_tab, src3, dst3, w1p, n_nodes, ch):
    nchunk = src3.shape[1]
    rpt = n_nodes // _NS

    mesh = plsc.VectorSubcoreMesh(core_axis_name="c", subcore_axis_name="s",
                                  num_cores=_NC, num_subcores=_NS)

    @functools.partial(
        pl.kernel,
        out_type=jax.ShapeDtypeStruct((_NW, rpt, _CHP), jnp.float32),
        mesh=mesh,
        compiler_params=pltpu.CompilerParams(use_tc_tiling_on_sc=False),
        scratch_types=[
            pltpu.VMEM((nchunk, _K), jnp.int32),
            pltpu.VMEM((nchunk, _K), jnp.int32),
            pltpu.VMEM((2, _K, _CHP), jnp.float32),
            pltpu.VMEM((2, _K, _CHP), jnp.float32),
            pltpu.VMEM((2, _K, _CHP), jnp.float32),
            pltpu.VMEM((_CHP, _CHP), jnp.float32),
            pltpu.VMEM((rpt, _CHP), jnp.float32),
            pltpu.VMEM_SHARED((n_nodes, _CHP), jnp.float32),
            pltpu.SemaphoreType.DMA,
            pltpu.SemaphoreType.DMA,
            pltpu.SemaphoreType.DMA,
            pltpu.SemaphoreType.DMA,
            pltpu.SemaphoreType.DMA,
            pltpu.SemaphoreType.DMA,
        ],
    )
    def edge_kernel(a_hbm, b_hbm, src_hbm, dst_hbm, w1_hbm, out_hbm,
                    sidx, didx, abuf, bbuf, mbuf, w1v, zbuf, agg,
                    sem_a0, sem_b0, sem_a1, sem_b1, sem_s0, sem_s1):
        sems = ((sem_a0, sem_b0), (sem_a1, sem_b1))
        ssems = (sem_s0, sem_s1)
        cid = lax.axis_index("c")
        tid = lax.axis_index("s")
        wid = cid * _NS + tid

        zv = jnp.zeros((_L,), jnp.float32)

        def _zero_z(i, carry):
            zbuf[i, :] = zv
            return carry

        lax.fori_loop(0, rpt, _zero_z, 0)

        def _zero_m(i, carry):
            mbuf[0, i, :] = zv
            mbuf[1, i, :] = zv
            return carry

        lax.fori_loop(0, _K, _zero_m, 0)

        pltpu.sync_copy(zbuf, agg.at[pl.ds(tid * rpt, rpt)])
        pltpu.sync_copy(w1_hbm, w1v)
        pltpu.sync_copy(src_hbm.at[wid], sidx)
        pltpu.sync_copy(dst_hbm.at[wid], didx)
        plsc.subcore_barrier()

        w1r = [w1v[c, :] for c in range(ch)]
        bcidx = [jnp.full((_L,), c, jnp.int32) for c in range(ch)]

        def issue_gathers(g, slot):
            pltpu.async_copy(a_hbm.at[didx.at[g]], abuf.at[slot], sems[slot][0])
            pltpu.async_copy(b_hbm.at[sidx.at[g]], bbuf.at[slot], sems[slot][1])

        def wait_gathers(g, slot):
            pltpu.make_async_copy(a_hbm.at[didx.at[g]], abuf.at[slot],
                                  sems[slot][0]).wait()
            pltpu.make_async_copy(b_hbm.at[sidx.at[g]], bbuf.at[slot],
                                  sems[slot][1]).wait()

        def wait_scatter(slot):
            pltpu.make_async_copy(mbuf.at[slot], agg.at[didx.at[0]],
                                  ssems[slot]).wait()

        def compute_scatter(g, slot):
            @plsc.parallel_loop(0, _K, unroll=_UNROLL)
            def edge_body(e):
                m = jnp.maximum(abuf[slot, e, :] + bbuf[slot, e, :], 0.0)
                acc = m.at[bcidx[0]].get(mode="promise_in_bounds") * w1r[0]
                for ci in range(1, ch):
                    acc = acc + m.at[bcidx[ci]].get(
                        mode="promise_in_bounds") * w1r[ci]
                mbuf[slot, e, :] = jnp.maximum(acc, 0.0)

            pltpu.async_copy(mbuf.at[slot], agg.at[didx.at[g]], ssems[slot],
                             add=True)

        issue_gathers(0, 0)
        pltpu.async_copy(mbuf.at[0], agg.at[didx.at[0]], ssems[0], add=True)
        pltpu.async_copy(mbuf.at[1], agg.at[didx.at[0]], ssems[1], add=True)

        def chunk_pair(i, carry):
            g0 = 2 * i
            issue_gathers(g0 + 1, 1)
            wait_gathers(g0, 0)
            wait_scatter(0)
            compute_scatter(g0, 0)
            issue_gathers(g0 + 2, 0)
            wait_gathers(g0 + 1, 1)
            wait_scatter(1)
            compute_scatter(g0 + 1, 1)
            return carry

        lax.fori_loop(0, (nchunk - 1) // 2, chunk_pair, 0)
        wait_gathers(nchunk - 1, 0)
        wait_scatter(0)
        compute_scatter(nchunk - 1, 0)
        wait_scatter(0)
        wait_scatter(1)

        plsc.subcore_barrier()
        pltpu.sync_copy(agg.at[pl.ds(tid * rpt, rpt)], out_hbm.at[wid])

    return edge_kernel(a_tab, b_tab, src3, dst3, w1p)


def _tc_pre(x, wt, wb):
    n = x.shape[0]

    def body(x_ref, wt_ref, wb_ref, a_ref, b_ref):
        xv = x_ref[...]
        a_ref[...] = jnp.dot(xv, wt_ref[...], preferred_element_type=jnp.float32)
        b_ref[...] = jnp.dot(xv, wb_ref[...], preferred_element_type=jnp.float32)

    return pl.pallas_call(
        body,
        out_shape=[jax.ShapeDtypeStruct((n, _CHP), jnp.float32)] * 2,
    )(x, wt, wb)


def _tc_mid(parts, wt, wb):
    n = parts.shape[1]

    def body(p_ref, wt_ref, wb_ref, a_ref, b_ref):
        h = jnp.maximum(p_ref[0] + p_ref[1], 0.0)
        a_ref[...] = jnp.dot(h, wt_ref[...], preferred_element_type=jnp.float32)
        b_ref[...] = jnp.dot(h, wb_ref[...], preferred_element_type=jnp.float32)

    return pl.pallas_call(
        body,
        out_shape=[jax.ShapeDtypeStruct((n, _CHP), jnp.float32)] * 2,
    )(parts, wt, wb)


def _tc_fin(parts, wf1p, wf2):
    n = parts.shape[1]
    feat = wf2.shape[1]

    def body(p_ref, w1_ref, w2_ref, o_ref):
        h = jnp.maximum(p_ref[0] + p_ref[1], 0.0)
        t = jnp.maximum(jnp.dot(h, w1_ref[...], preferred_element_type=jnp.float32), 0.0)
        o_ref[...] = jnp.maximum(jnp.dot(t, w2_ref[...], preferred_element_type=jnp.float32), 0.0)

    return pl.pallas_call(
        body,
        out_shape=jax.ShapeDtypeStruct((n, feat), jnp.float32),
    )(parts, wf1p, wf2)


def kernel(x, edge_index, edge_attr, W2_0, W1_0, W2_1, W1_1, Wf1, Wf2):
    del edge_attr
    n, feat = x.shape
    ch = W1_0.shape[0]
    pad_c = _CHP - ch

    n_edges = edge_index.shape[1]
    nchunk = n_edges // (_NW * _K)
    src3 = edge_index[0].astype(jnp.int32).reshape(_NW, nchunk, _K)
    dst3 = edge_index[1].astype(jnp.int32).reshape(_NW, nchunk, _K)

    wt0 = jnp.pad(W2_0[:feat], ((0, 0), (0, pad_c)))
    wb0 = jnp.pad(W2_0[feat:], ((0, 0), (0, pad_c)))
    w1p0 = jnp.pad(W1_0 * 0.1, ((0, pad_c), (0, pad_c)))
    wt1 = jnp.pad(W2_1[:ch], ((0, pad_c), (0, pad_c)))
    wb1 = jnp.pad(W2_1[ch:], ((0, pad_c), (0, pad_c)))
    w1p1 = jnp.pad(W1_1 * 0.1, ((0, pad_c), (0, pad_c)))
    wf1p = jnp.pad(Wf1, ((0, pad_c), (0, 0)))

    a0, b0 = _tc_pre(x, wt0, wb0)
    parts0 = _sc_edge_pass(a0, b0, src3, dst3, w1p0, n, ch)
    a1, b1 = _tc_mid(parts0.reshape(_NC, n, _CHP), wt1, wb1)
    parts1 = _sc_edge_pass(a1, b1, src3, dst3, w1p1, n, ch)
    return _tc_fin(parts1.reshape(_NC, n, _CHP), wf1p, Wf2)


# speedup vs baseline: 2.3536x; 1.0013x over previous
import functools

import jax
import jax.numpy as jnp
from jax import lax
from jax.experimental import pallas as pl
from jax.experimental.pallas import tpu as pltpu
from jax.experimental.pallas import tpu_sc as plsc

_NC = 2
_NS = 16
_NW = _NC * _NS
_L = 16
_CHP = 16
_K = 80
_UNROLL = 10


def _sc_edge_pass(a_tab, b_tab, src3, dst3, w1p, n_nodes, ch):
    nchunk = src3.shape[1]
    rpt = n_nodes // _NS

    mesh = plsc.VectorSubcoreMesh(core_axis_name="c", subcore_axis_name="s",
                                  num_cores=_NC, num_subcores=_NS)

    @functools.partial(
        pl.kernel,
        out_type=jax.ShapeDtypeStruct((_NW, rpt, _CHP), jnp.float32),
        mesh=mesh,
        compiler_params=pltpu.CompilerParams(use_tc_tiling_on_sc=False),
        scratch_types=[
            pltpu.VMEM((nchunk, _K), jnp.int32),
            pltpu.VMEM((nchunk, _K), jnp.int32),
            pltpu.VMEM((2, _K, _CHP), jnp.float32),
            pltpu.VMEM((2, _K, _CHP), jnp.float32),
            pltpu.VMEM((2, _K, _CHP), jnp.float32),
            pltpu.VMEM((_CHP, _CHP), jnp.float32),
            pltpu.VMEM((rpt, _CHP), jnp.float32),
            pltpu.VMEM_SHARED((n_nodes, _CHP), jnp.float32),
            pltpu.SemaphoreType.DMA,
            pltpu.SemaphoreType.DMA,
            pltpu.SemaphoreType.DMA,
            pltpu.SemaphoreType.DMA,
            pltpu.SemaphoreType.DMA,
            pltpu.SemaphoreType.DMA,
        ],
    )
    def edge_kernel(a_hbm, b_hbm, src_hbm, dst_hbm, w1_hbm, out_hbm,
                    sidx, didx, abuf, bbuf, mbuf, w1v, zbuf, agg,
                    sem_a0, sem_b0, sem_a1, sem_b1, sem_s0, sem_s1):
        sems = ((sem_a0, sem_b0), (sem_a1, sem_b1))
        ssems = (sem_s0, sem_s1)
        cid = lax.axis_index("c")
        tid = lax.axis_index("s")
        wid = cid * _NS + tid

        zv = jnp.zeros((_L,), jnp.float32)

        def _zero_z(i, carry):
            zbuf[i, :] = zv
            return carry

        lax.fori_loop(0, rpt, _zero_z, 0)

        def _zero_m(i, carry):
            mbuf[0, i, :] = zv
            mbuf[1, i, :] = zv
            return carry

        lax.fori_loop(0, _K, _zero_m, 0)

        pltpu.sync_copy(zbuf, agg.at[pl.ds(tid * rpt, rpt)])
        pltpu.sync_copy(w1_hbm, w1v)
        pltpu.sync_copy(src_hbm.at[wid], sidx)
        pltpu.sync_copy(dst_hbm.at[wid], didx)
        plsc.subcore_barrier()

        w1r = [w1v[c, :] for c in range(ch)]
        bcidx = [jnp.full((_L,), c, jnp.int32) for c in range(ch)]

        def issue_gathers(g, slot):
            pltpu.async_copy(a_hbm.at[didx.at[g]], abuf.at[slot], sems[slot][0])
            pltpu.async_copy(b_hbm.at[sidx.at[g]], bbuf.at[slot], sems[slot][1])

        def wait_gathers(g, slot):
            pltpu.make_async_copy(a_hbm.at[didx.at[g]], abuf.at[slot],
                                  sems[slot][0]).wait()
            pltpu.make_async_copy(b_hbm.at[sidx.at[g]], bbuf.at[slot],
                                  sems[slot][1]).wait()

        def wait_scatter(slot):
            pltpu.make_async_copy(mbuf.at[slot], agg.at[didx.at[0]],
                                  ssems[slot]).wait()

        def compute_scatter(g, slot):
            @plsc.parallel_loop(0, _K, unroll=_UNROLL)
            def edge_body(e):
                m = jnp.maximum(abuf[slot, e, :] + bbuf[slot, e, :], 0.0)
                acc = m.at[bcidx[0]].get(mode="promise_in_bounds") * w1r[0]
                for ci in range(1, ch):
                    acc = acc + m.at[bcidx[ci]].get(
                        mode="promise_in_bounds") * w1r[ci]
                mbuf[slot, e, :] = jnp.maximum(acc, 0.0)

            pltpu.async_copy(mbuf.at[slot], agg.at[didx.at[g]], ssems[slot],
                             add=True)

        issue_gathers(0, 0)
        pltpu.async_copy(mbuf.at[0], agg.at[didx.at[0]], ssems[0], add=True)
        pltpu.async_copy(mbuf.at[1], agg.at[didx.at[0]], ssems[1], add=True)

        def chunk_pair(i, carry):
            g0 = 2 * i
            issue_gathers(g0 + 1, 1)
            wait_gathers(g0, 0)
            wait_scatter(0)
            compute_scatter(g0, 0)
            issue_gathers(g0 + 2, 0)
            wait_gathers(g0 + 1, 1)
            wait_scatter(1)
            compute_scatter(g0 + 1, 1)
            return carry

        lax.fori_loop(0, (nchunk - 1) // 2, chunk_pair, 0)
        wait_gathers(nchunk - 1, 0)
        wait_scatter(0)
        compute_scatter(nchunk - 1, 0)
        wait_scatter(0)
        wait_scatter(1)

        plsc.subcore_barrier()
        pltpu.sync_copy(agg.at[pl.ds(tid * rpt, rpt)], out_hbm.at[wid])

    return edge_kernel(a_tab, b_tab, src3, dst3, w1p)


def _tc_pre(x, wt, wb):
    n = x.shape[0]

    def body(x_ref, wt_ref, wb_ref, a_ref, b_ref):
        xv = x_ref[...]
        a_ref[...] = jnp.dot(xv, wt_ref[...], preferred_element_type=jnp.float32)
        b_ref[...] = jnp.dot(xv, wb_ref[...], preferred_element_type=jnp.float32)

    return pl.pallas_call(
        body,
        out_shape=[jax.ShapeDtypeStruct((n, _CHP), jnp.float32)] * 2,
    )(x, wt, wb)


def _tc_mid(parts, wt, wb):
    n = parts.shape[1]

    def body(p_ref, wt_ref, wb_ref, a_ref, b_ref):
        h = jnp.maximum(p_ref[0] + p_ref[1], 0.0)
        a_ref[...] = jnp.dot(h, wt_ref[...], preferred_element_type=jnp.float32)
        b_ref[...] = jnp.dot(h, wb_ref[...], preferred_element_type=jnp.float32)

    return pl.pallas_call(
        body,
        out_shape=[jax.ShapeDtypeStruct((n, _CHP), jnp.float32)] * 2,
    )(parts, wt, wb)


def _tc_fin(parts, wf1p, wf2):
    n = parts.shape[1]
    feat = wf2.shape[1]

    def body(p_ref, w1_ref, w2_ref, o_ref):
        h = jnp.maximum(p_ref[0] + p_ref[1], 0.0)
        t = jnp.maximum(jnp.dot(h, w1_ref[...], preferred_element_type=jnp.float32), 0.0)
        o_ref[...] = jnp.maximum(jnp.dot(t, w2_ref[...], preferred_element_type=jnp.float32), 0.0)

    return pl.pallas_call(
        body,
        out_shape=jax.ShapeDtypeStruct((n, feat), jnp.float32),
    )(parts, wf1p, wf2)


def kernel(x, edge_index, edge_attr, W2_0, W1_0, W2_1, W1_1, Wf1, Wf2):
    del edge_attr
    n, feat = x.shape
    ch = W1_0.shape[0]
    pad_c = _CHP - ch

    n_edges = edge_index.shape[1]
    nchunk = n_edges // (_NW * _K)
    src3 = edge_index[0].astype(jnp.int32).reshape(_NW, nchunk, _K)
    dst3 = edge_index[1].astype(jnp.int32).reshape(_NW, nchunk, _K)

    wt0 = jnp.pad(W2_0[:feat], ((0, 0), (0, pad_c)))
    wb0 = jnp.pad(W2_0[feat:], ((0, 0), (0, pad_c)))
    w1p0 = jnp.pad(W1_0 * 0.1, ((0, pad_c), (0, pad_c)))
    wt1 = jnp.pad(W2_1[:ch], ((0, pad_c), (0, pad_c)))
    wb1 = jnp.pad(W2_1[ch:], ((0, pad_c), (0, pad_c)))
    w1p1 = jnp.pad(W1_1 * 0.1, ((0, pad_c), (0, pad_c)))
    wf1p = jnp.pad(Wf1, ((0, pad_c), (0, 0)))

    a0, b0 = _tc_pre(x, wt0, wb0)
    parts0 = _sc_edge_pass(a0, b0, src3, dst3, w1p0, n, ch)
    a1, b1 = _tc_mid(parts0.reshape(_NC, n, _CHP), wt1, wb1)
    parts1 = _sc_edge_pass(a1, b1, src3, dst3, w1p1, n, ch)
    return _tc_fin(parts1.reshape(_NC, n, _CHP), wf1p, Wf2)


# speedup vs baseline: 2.5987x; 1.1041x over previous
import functools

import jax
import jax.numpy as jnp
from jax import lax
from jax.experimental import pallas as pl
from jax.experimental.pallas import tpu as pltpu
from jax.experimental.pallas import tpu_sc as plsc

_NC = 2
_NS = 16
_NW = _NC * _NS
_L = 16
_CHP = 16
_K = 80
_UNROLL = 10


def _sc_edge_pass(a_tab, b_tab, src3, dst3, w1p, n_nodes, ch):
    nchunk = src3.shape[1]
    rpt = n_nodes // _NS

    mesh = plsc.VectorSubcoreMesh(core_axis_name="c", subcore_axis_name="s",
                                  num_cores=_NC, num_subcores=_NS)

    @functools.partial(
        pl.kernel,
        out_type=jax.ShapeDtypeStruct((_NW, rpt, _CHP), jnp.float32),
        mesh=mesh,
        compiler_params=pltpu.CompilerParams(use_tc_tiling_on_sc=False),
        scratch_types=[
            pltpu.VMEM((nchunk, _K), jnp.int32),
            pltpu.VMEM((nchunk, _K), jnp.int32),
            pltpu.VMEM((2, _K, _CHP), jnp.float32),
            pltpu.VMEM((2, _K, _CHP), jnp.float32),
            pltpu.VMEM((2, _K, _CHP), jnp.float32),
            pltpu.VMEM((_CHP, _CHP), jnp.float32),
            pltpu.VMEM((rpt, _CHP), jnp.float32),
            pltpu.VMEM_SHARED((n_nodes, _CHP), jnp.float32),
            pltpu.VMEM_SHARED((n_nodes, _CHP), jnp.float32),
            pltpu.VMEM_SHARED((n_nodes, _CHP), jnp.float32),
            pltpu.SemaphoreType.DMA,
            pltpu.SemaphoreType.DMA,
            pltpu.SemaphoreType.DMA,
            pltpu.SemaphoreType.DMA,
            pltpu.SemaphoreType.DMA,
            pltpu.SemaphoreType.DMA,
        ],
    )
    def edge_kernel(a_hbm, b_hbm, src_hbm, dst_hbm, w1_hbm, out_hbm,
                    sidx, didx, abuf, bbuf, mbuf, w1v, zbuf, agg, a_sp, b_sp,
                    sem_a0, sem_b0, sem_a1, sem_b1, sem_s0, sem_s1):
        sems = ((sem_a0, sem_b0), (sem_a1, sem_b1))
        ssems = (sem_s0, sem_s1)
        cid = lax.axis_index("c")
        tid = lax.axis_index("s")
        wid = cid * _NS + tid

        zv = jnp.zeros((_L,), jnp.float32)

        def _zero_z(i, carry):
            zbuf[i, :] = zv
            return carry

        lax.fori_loop(0, rpt, _zero_z, 0)

        def _zero_m(i, carry):
            mbuf[0, i, :] = zv
            mbuf[1, i, :] = zv
            return carry

        lax.fori_loop(0, _K, _zero_m, 0)

        pltpu.sync_copy(zbuf, agg.at[pl.ds(tid * rpt, rpt)])
        pltpu.sync_copy(w1_hbm, w1v)
        pltpu.sync_copy(src_hbm.at[wid], sidx)
        pltpu.sync_copy(dst_hbm.at[wid], didx)
        pltpu.sync_copy(a_hbm.at[pl.ds(tid * rpt, rpt)],
                        a_sp.at[pl.ds(tid * rpt, rpt)])
        pltpu.sync_copy(b_hbm.at[pl.ds(tid * rpt, rpt)],
                        b_sp.at[pl.ds(tid * rpt, rpt)])
        plsc.subcore_barrier()

        w1r = [w1v[c, :] for c in range(ch)]
        bcidx = [jnp.full((_L,), c, jnp.int32) for c in range(ch)]

        def issue_gathers(g, slot):
            pltpu.async_copy(a_sp.at[didx.at[g]], abuf.at[slot], sems[slot][0])
            pltpu.async_copy(b_sp.at[sidx.at[g]], bbuf.at[slot], sems[slot][1])

        def wait_gathers(g, slot):
            pltpu.make_async_copy(a_sp.at[didx.at[g]], abuf.at[slot],
                                  sems[slot][0]).wait()
            pltpu.make_async_copy(b_sp.at[sidx.at[g]], bbuf.at[slot],
                                  sems[slot][1]).wait()

        def wait_scatter(slot):
            pltpu.make_async_copy(mbuf.at[slot], agg.at[didx.at[0]],
                                  ssems[slot]).wait()

        def compute_scatter(g, slot):
            @plsc.parallel_loop(0, _K, unroll=_UNROLL)
            def edge_body(e):
                m = jnp.maximum(abuf[slot, e, :] + bbuf[slot, e, :], 0.0)
                acc = m.at[bcidx[0]].get(mode="promise_in_bounds") * w1r[0]
                for ci in range(1, ch):
                    acc = acc + m.at[bcidx[ci]].get(
                        mode="promise_in_bounds") * w1r[ci]
                mbuf[slot, e, :] = jnp.maximum(acc, 0.0)

            pltpu.async_copy(mbuf.at[slot], agg.at[didx.at[g]], ssems[slot],
                             add=True)

        issue_gathers(0, 0)
        pltpu.async_copy(mbuf.at[0], agg.at[didx.at[0]], ssems[0], add=True)
        pltpu.async_copy(mbuf.at[1], agg.at[didx.at[0]], ssems[1], add=True)

        def chunk_pair(i, carry):
            g0 = 2 * i
            issue_gathers(g0 + 1, 1)
            wait_gathers(g0, 0)
            wait_scatter(0)
            compute_scatter(g0, 0)
            issue_gathers(g0 + 2, 0)
            wait_gathers(g0 + 1, 1)
            wait_scatter(1)
            compute_scatter(g0 + 1, 1)
            return carry

        lax.fori_loop(0, (nchunk - 1) // 2, chunk_pair, 0)
        wait_gathers(nchunk - 1, 0)
        wait_scatter(0)
        compute_scatter(nchunk - 1, 0)
        wait_scatter(0)
        wait_scatter(1)

        plsc.subcore_barrier()
        pltpu.sync_copy(agg.at[pl.ds(tid * rpt, rpt)], out_hbm.at[wid])

    return edge_kernel(a_tab, b_tab, src3, dst3, w1p)


def _tc_pre(x, wt, wb):
    n = x.shape[0]

    def body(x_ref, wt_ref, wb_ref, a_ref, b_ref):
        xv = x_ref[...]
        a_ref[...] = jnp.dot(xv, wt_ref[...], preferred_element_type=jnp.float32)
        b_ref[...] = jnp.dot(xv, wb_ref[...], preferred_element_type=jnp.float32)

    return pl.pallas_call(
        body,
        out_shape=[jax.ShapeDtypeStruct((n, _CHP), jnp.float32)] * 2,
    )(x, wt, wb)


def _tc_mid(parts, wt, wb):
    n = parts.shape[1]

    def body(p_ref, wt_ref, wb_ref, a_ref, b_ref):
        h = jnp.maximum(p_ref[0] + p_ref[1], 0.0)
        a_ref[...] = jnp.dot(h, wt_ref[...], preferred_element_type=jnp.float32)
        b_ref[...] = jnp.dot(h, wb_ref[...], preferred_element_type=jnp.float32)

    return pl.pallas_call(
        body,
        out_shape=[jax.ShapeDtypeStruct((n, _CHP), jnp.float32)] * 2,
    )(parts, wt, wb)


def _tc_fin(parts, wf1p, wf2):
    n = parts.shape[1]
    feat = wf2.shape[1]

    def body(p_ref, w1_ref, w2_ref, o_ref):
        h = jnp.maximum(p_ref[0] + p_ref[1], 0.0)
        t = jnp.maximum(jnp.dot(h, w1_ref[...], preferred_element_type=jnp.float32), 0.0)
        o_ref[...] = jnp.maximum(jnp.dot(t, w2_ref[...], preferred_element_type=jnp.float32), 0.0)

    return pl.pallas_call(
        body,
        out_shape=jax.ShapeDtypeStruct((n, feat), jnp.float32),
    )(parts, wf1p, wf2)


def kernel(x, edge_index, edge_attr, W2_0, W1_0, W2_1, W1_1, Wf1, Wf2):
    del edge_attr
    n, feat = x.shape
    ch = W1_0.shape[0]
    pad_c = _CHP - ch

    n_edges = edge_index.shape[1]
    nchunk = n_edges // (_NW * _K)
    src3 = edge_index[0].astype(jnp.int32).reshape(_NW, nchunk, _K)
    dst3 = edge_index[1].astype(jnp.int32).reshape(_NW, nchunk, _K)

    wt0 = jnp.pad(W2_0[:feat], ((0, 0), (0, pad_c)))
    wb0 = jnp.pad(W2_0[feat:], ((0, 0), (0, pad_c)))
    w1p0 = jnp.pad(W1_0 * 0.1, ((0, pad_c), (0, pad_c)))
    wt1 = jnp.pad(W2_1[:ch], ((0, pad_c), (0, pad_c)))
    wb1 = jnp.pad(W2_1[ch:], ((0, pad_c), (0, pad_c)))
    w1p1 = jnp.pad(W1_1 * 0.1, ((0, pad_c), (0, pad_c)))
    wf1p = jnp.pad(Wf1, ((0, pad_c), (0, 0)))

    a0, b0 = _tc_pre(x, wt0, wb0)
    parts0 = _sc_edge_pass(a0, b0, src3, dst3, w1p0, n, ch)
    a1, b1 = _tc_mid(parts0.reshape(_NC, n, _CHP), wt1, wb1)
    parts1 = _sc_edge_pass(a1, b1, src3, dst3, w1p1, n, ch)
    return _tc_fin(parts1.reshape(_NC, n, _CHP), wf1p, Wf2)
